# Initial kernel scaffold; baseline (speedup 1.0000x reference)
import jax
import jax.numpy as jnp
from jax.experimental import pallas as pl


def kernel(t, y, edge_index, edge_type, W1, b1, rel1, W2, b2, rel2, res, Wmu, bmu):
    raise NotImplementedError("write your pallas kernel here")


# speedup vs baseline: 6.5501x; 6.5501x over previous
import functools

import jax
import jax.numpy as jnp
from jax import lax
from jax.experimental import pallas as pl
from jax.experimental.pallas import tpu as pltpu
from jax.experimental.pallas import tpu_sc as plsc

N_NODES = 10000
DIM = 128
N_EDGES = 320000
N_REL = 16

NC = 2
NS = 16
NW = NC * NS

CHUNK = 128
KCH = 79
PW = KCH * CHUNK
E_PAD = PW * NW
NP = 10112
STRIPE = NP // NS

_mesh = plsc.VectorSubcoreMesh(core_axis_name="c", subcore_axis_name="s",
                               num_cores=NC, num_subcores=NS)


def _sc_edge_body(xr_hbm, idx_hbm, dst_hbm, zeros_hbm, agg_out, agg_sh,
                  idx_v, dst_v, buf, sem_g):
    c = lax.axis_index("c")
    s = lax.axis_index("s")
    wid = c * NS + s

    pltpu.sync_copy(idx_hbm.at[wid], idx_v)
    pltpu.sync_copy(dst_hbm.at[wid], dst_v)
    row0 = s * STRIPE
    pltpu.sync_copy(zeros_hbm.at[pl.ds(row0, STRIPE)],
                    agg_sh.at[pl.ds(row0, STRIPE)])
    plsc.subcore_barrier()

    @pl.loop(0, KCH)
    def _(j):
        pltpu.async_copy(xr_hbm.at[idx_v.at[j]], buf, sem_g).wait()
        pltpu.sync_copy(buf, agg_sh.at[dst_v.at[j]], add=True)

    plsc.subcore_barrier()
    pltpu.sync_copy(agg_sh.at[pl.ds(row0, STRIPE)],
                    agg_out.at[c, pl.ds(row0, STRIPE)])


_sc_edges = pl.kernel(
    _sc_edge_body,
    out_type=jax.ShapeDtypeStruct((NC, NP, DIM), jnp.float32),
    mesh=_mesh,
    scratch_types=(
        pltpu.VMEM_SHARED((NP, DIM), jnp.float32),
        pltpu.VMEM((KCH, CHUNK), jnp.int32),
        pltpu.VMEM((KCH, CHUNK), jnp.int32),
        pltpu.VMEM((CHUNK, DIM), jnp.float32),
        pltpu.SemaphoreType.DMA,
    ),
)


def _sc_deg_body(dst_hbm, zdeg_hbm, ones_hbm, deg_out, deg_sh, dst_v,
                 ones_v):
    c = lax.axis_index("c")
    s = lax.axis_index("s")
    wid = c * NS + s

    pltpu.sync_copy(dst_hbm.at[wid], dst_v)
    pltpu.sync_copy(ones_hbm, ones_v)
    row0 = s * STRIPE
    pltpu.sync_copy(zdeg_hbm.at[pl.ds(row0, STRIPE)],
                    deg_sh.at[pl.ds(row0, STRIPE)])
    plsc.subcore_barrier()

    @pl.loop(0, KCH)
    def _(j):
        pltpu.sync_copy(ones_v, deg_sh.at[dst_v.at[j]], add=True)

    plsc.subcore_barrier()
    pltpu.sync_copy(deg_sh.at[pl.ds(row0, STRIPE)],
                    deg_out.at[c, pl.ds(row0, STRIPE)])


_sc_deg = pl.kernel(
    _sc_deg_body,
    out_type=jax.ShapeDtypeStruct((NC, NP, DIM), jnp.float32),
    mesh=_mesh,
    scratch_types=(
        pltpu.VMEM_SHARED((NP, DIM), jnp.float32),
        pltpu.VMEM((KCH, CHUNK), jnp.int32),
        pltpu.VMEM((CHUNK, DIM), jnp.float32),
    ),
)


_EROWS = E_PAD // 128


def _eidx_body(src_ref, et_ref, out_ref):
    out_ref[...] = et_ref[...] * N_NODES + src_ref[...]


def _edge_idx(src_p, et_p):
    return pl.pallas_call(
        _eidx_body,
        out_shape=jax.ShapeDtypeStruct((_EROWS, 128), jnp.int32),
    )(src_p.reshape(_EROWS, 128), et_p.reshape(_EROWS, 128))


NB = 400


def _xr_body(x_ref, rel_ref, out_ref):
    out_ref[...] = rel_ref[...][:, None, :] * x_ref[...][None, :, :]


def _build_xr(x, rel):
    grid = (N_NODES // NB,)
    xr = pl.pallas_call(
        _xr_body,
        grid=grid,
        in_specs=[
            pl.BlockSpec((NB, DIM), lambda i: (i, 0)),
            pl.BlockSpec((N_REL, DIM), lambda i: (0, 0)),
        ],
        out_specs=pl.BlockSpec((N_REL, NB, DIM), lambda i: (0, i, 0)),
        out_shape=jax.ShapeDtypeStruct((N_REL, N_NODES, DIM), jnp.float32),
    )(x, rel)
    return xr.reshape(N_REL * N_NODES, DIM)


def _post_body(final, parts_ref, degp_ref, x_ref, W_ref, b_ref, res_ref,
               W2_ref, b2_ref, out_ref):
    deg = degp_ref[0, :, 0:1] + degp_ref[1, :, 0:1]
    deg = jnp.maximum(deg, 1.0)
    agg = (parts_ref[0] + parts_ref[1]) / deg
    h = jax.nn.relu(
        jnp.dot(agg, W_ref[...], preferred_element_type=jnp.float32)
        + b_ref[...])
    emb = x_ref[...] + res_ref[0, 0] * h
    if final:
        out_ref[...] = (
            jnp.dot(emb, W2_ref[...], preferred_element_type=jnp.float32)
            + b2_ref[...])
    else:
        out_ref[...] = emb


def _post(final, parts, degp, x, W, b, res, W2, b2):
    grid = (N_NODES // NB,)
    return pl.pallas_call(
        functools.partial(_post_body, final),
        grid=grid,
        in_specs=[
            pl.BlockSpec((NC, NB, DIM), lambda i: (0, i, 0)),
            pl.BlockSpec((NC, NB, DIM), lambda i: (0, i, 0)),
            pl.BlockSpec((NB, DIM), lambda i: (i, 0)),
            pl.BlockSpec((DIM, DIM), lambda i: (0, 0)),
            pl.BlockSpec((1, DIM), lambda i: (0, 0)),
            pl.BlockSpec((1, 1), lambda i: (0, 0)),
            pl.BlockSpec((DIM, DIM), lambda i: (0, 0)),
            pl.BlockSpec((1, DIM), lambda i: (0, 0)),
        ],
        out_specs=pl.BlockSpec((NB, DIM), lambda i: (i, 0)),
        out_shape=jax.ShapeDtypeStruct((N_NODES, DIM), jnp.float32),
    )(parts, degp, x, W, b, res, W2, b2)


def kernel(t, y, edge_index, edge_type, W1, b1, rel1, W2, b2, rel2, res,
           Wmu, bmu):
    del t
    pad = E_PAD - N_EDGES
    src_p = jnp.concatenate(
        [edge_index[0], jnp.zeros((pad,), edge_index.dtype)]).astype(jnp.int32)
    et_p = jnp.concatenate(
        [edge_type, jnp.zeros((pad,), edge_type.dtype)]).astype(jnp.int32)
    dst_p = jnp.concatenate(
        [edge_index[1],
         jnp.full((pad,), N_NODES, edge_index.dtype)]).astype(jnp.int32)

    idx = _edge_idx(src_p, et_p).reshape(NW, KCH, CHUNK)
    dst3 = dst_p.reshape(NW, KCH, CHUNK)

    zeros_big = jnp.zeros((NP, DIM), jnp.float32)

    ones_src = jnp.ones((CHUNK, DIM), jnp.float32)

    b1r = b1.reshape(1, DIM)
    b2r = b2.reshape(1, DIM)
    bmur = bmu.reshape(1, DIM)
    resr = res.reshape(1, 1)

    degp = _sc_deg(dst3, zeros_big, ones_src)
    xr1 = _build_xr(y, rel1)
    agg1 = _sc_edges(xr1, idx, dst3, zeros_big)
    emb1 = _post(False, agg1, degp, y, W1, b1r, resr, W1, b1r)

    xr2 = _build_xr(emb1, rel2)
    agg2 = _sc_edges(xr2, idx, dst3, zeros_big)
    out = _post(True, agg2, degp, emb1, W2, b2r, resr, Wmu, bmur)
    return out
